# baseline (speedup 1.0000x reference)
import functools

import jax
import jax.numpy as jnp
from jax import lax
from jax.experimental import pallas as pl
from jax.experimental.pallas import tpu as pltpu
from jax.experimental.pallas import tpu_sc as plsc

BATCH = 1024
N_HIDDEN = 16
N_FEATURES = 100000

_NC = 2
_NS = 16
_NW = _NC * _NS
_B_PER_W = BATCH // _NW


def _sc_gather(data, emb):
    mesh = plsc.VectorSubcoreMesh(core_axis_name="c", subcore_axis_name="s")

    @functools.partial(
        pl.kernel,
        mesh=mesh,
        out_type=jax.ShapeDtypeStruct((BATCH, N_HIDDEN), jnp.float32),
        scratch_types=[
            pltpu.VMEM((_B_PER_W,), jnp.int32),
            pltpu.VMEM((_B_PER_W, N_HIDDEN), jnp.float32),
            pltpu.SemaphoreType.DMA,
        ],
        compiler_params=pltpu.CompilerParams(use_tc_tiling_on_sc=False),
    )
    def gather_kernel(idx_hbm, table_hbm, out_hbm, idx_v, rows_v, sem):
        wid = lax.axis_index("s") * _NC + lax.axis_index("c")
        base = wid * _B_PER_W
        pltpu.sync_copy(idx_hbm.at[pl.ds(base, _B_PER_W)], idx_v)
        pltpu.async_copy(table_hbm.at[idx_v], rows_v, sem).wait()
        pltpu.sync_copy(rows_v, out_hbm.at[pl.ds(base, _B_PER_W)])

    return gather_kernel(data, emb)


_TJ = 1024


def _proj_kernel(x_ref, w_ref, b_ref, out_ref):
    acc = lax.dot_general(
        x_ref[...], w_ref[...],
        (((1,), (1,)), ((), ())),
        preferred_element_type=jnp.float32,
    )
    out_ref[...] = acc + b_ref[...]


def _tc_project(x, W, b):
    nj = pl.cdiv(N_FEATURES, _TJ)
    return pl.pallas_call(
        _proj_kernel,
        grid=(nj,),
        in_specs=[
            pl.BlockSpec((BATCH, N_HIDDEN), lambda j: (0, 0)),
            pl.BlockSpec((_TJ, N_HIDDEN), lambda j: (j, 0)),
            pl.BlockSpec((1, _TJ), lambda j: (0, j)),
        ],
        out_specs=pl.BlockSpec((BATCH, _TJ), lambda j: (0, j)),
        out_shape=jax.ShapeDtypeStruct((BATCH, N_FEATURES), jnp.float32),
    )(x, W, b)


def kernel(data, emb, W, b):
    x = _sc_gather(data, emb)
    return _tc_project(x, W, b[None, :])


# speedup vs baseline: 1.0389x; 1.0389x over previous
import functools

import jax
import jax.numpy as jnp
from jax import lax
from jax.experimental import pallas as pl
from jax.experimental.pallas import tpu as pltpu
from jax.experimental.pallas import tpu_sc as plsc

BATCH = 1024
N_HIDDEN = 16
N_FEATURES = 100000

_NC = 2
_NS = 16
_NW = _NC * _NS
_B_PER_W = BATCH // _NW


def _sc_gather(data, emb):
    mesh = plsc.VectorSubcoreMesh(core_axis_name="c", subcore_axis_name="s")

    @functools.partial(
        pl.kernel,
        mesh=mesh,
        out_type=jax.ShapeDtypeStruct((BATCH, N_HIDDEN), jnp.float32),
        scratch_types=[
            pltpu.VMEM((_B_PER_W,), jnp.int32),
            pltpu.VMEM((_B_PER_W, N_HIDDEN), jnp.float32),
            pltpu.SemaphoreType.DMA,
        ],
        compiler_params=pltpu.CompilerParams(use_tc_tiling_on_sc=False),
    )
    def gather_kernel(idx_hbm, table_hbm, out_hbm, idx_v, rows_v, sem):
        wid = lax.axis_index("s") * _NC + lax.axis_index("c")
        base = wid * _B_PER_W
        pltpu.sync_copy(idx_hbm.at[pl.ds(base, _B_PER_W)], idx_v)
        pltpu.async_copy(table_hbm.at[idx_v], rows_v, sem).wait()
        pltpu.sync_copy(rows_v, out_hbm.at[pl.ds(base, _B_PER_W)])

    return gather_kernel(data, emb)


_TJ = 4096


def _proj_kernel(x_ref, w_ref, b_ref, out_ref):
    acc = lax.dot_general(
        x_ref[...], w_ref[...],
        (((1,), (1,)), ((), ())),
        preferred_element_type=jnp.float32,
    )
    out_ref[...] = acc + b_ref[...]


def _tc_project(x, W, b):
    nj = pl.cdiv(N_FEATURES, _TJ)
    return pl.pallas_call(
        _proj_kernel,
        grid=(nj,),
        in_specs=[
            pl.BlockSpec((BATCH, N_HIDDEN), lambda j: (0, 0)),
            pl.BlockSpec((_TJ, N_HIDDEN), lambda j: (j, 0)),
            pl.BlockSpec((1, _TJ), lambda j: (0, j)),
        ],
        out_specs=pl.BlockSpec((BATCH, _TJ), lambda j: (0, j)),
        out_shape=jax.ShapeDtypeStruct((BATCH, N_FEATURES), jnp.float32),
    )(x, W, b)


def kernel(data, emb, W, b):
    x = _sc_gather(data, emb)
    return _tc_project(x, W, b[None, :])
